# baseline (speedup 1.0000x reference)
import jax
import jax.numpy as jnp
from jax.experimental import pallas as pl
from jax.experimental.pallas import tpu as pltpu

_SIZE = 65536
_DIM = 128
_BATCH = 1024


def _body(out_hbm, bank_hbm, oo_hbm, snap_hbm, new_hbm, vin, vout, sems, osem):
    snap_cp = pltpu.make_async_copy(bank_hbm, snap_hbm, sems.at[0])
    new_cp = pltpu.make_async_copy(bank_hbm, new_hbm, sems.at[1])
    oo_cp = pltpu.make_async_copy(out_hbm, oo_hbm, sems.at[2])
    snap_cp.start()
    new_cp.start()
    oo_cp.start()
    in_cp = pltpu.make_async_copy(out_hbm, vin, osem)
    in_cp.start()
    in_cp.wait()
    vout[...] = jnp.transpose(vin[...])
    new_cp.wait()
    out_cp = pltpu.make_async_copy(vout, new_hbm.at[:, pl.ds(0, _BATCH)], osem)
    out_cp.start()
    out_cp.wait()
    snap_cp.wait()
    oo_cp.wait()


def kernel(output, bank):
    out_shapes = (
        jax.ShapeDtypeStruct((_BATCH, _DIM), output.dtype),
        jax.ShapeDtypeStruct((_DIM, _SIZE), bank.dtype),
        jax.ShapeDtypeStruct((_DIM, _SIZE), bank.dtype),
    )
    out, snap, new = pl.pallas_call(
        _body,
        in_specs=[
            pl.BlockSpec(memory_space=pl.ANY),
            pl.BlockSpec(memory_space=pl.ANY),
        ],
        out_specs=[
            pl.BlockSpec(memory_space=pl.ANY),
            pl.BlockSpec(memory_space=pl.ANY),
            pl.BlockSpec(memory_space=pl.ANY),
        ],
        out_shape=out_shapes,
        scratch_shapes=[
            pltpu.VMEM((_BATCH, _DIM), jnp.float32),
            pltpu.VMEM((_DIM, _BATCH), jnp.float32),
            pltpu.SemaphoreType.DMA((3,)),
            pltpu.SemaphoreType.DMA,
        ],
    )(output, bank)
    return (out, snap, new)


# speedup vs baseline: 33.2440x; 33.2440x over previous
import functools

import jax
import jax.numpy as jnp
from jax import lax
from jax.experimental import pallas as pl
from jax.experimental.pallas import tpu as pltpu
from jax.experimental.pallas import tpu_sc as plsc

_SIZE = 65536
_DIM = 128
_BATCH = 1024
_BLK = 16384
_GRID = _SIZE // _BLK

_NW = 32
_ROWS_PER_W = _DIM // _NW
_CHUNK = 16384
_CHUNKS_PER_ROW = _SIZE // _CHUNK
_TOTAL = _ROWS_PER_W * _CHUNKS_PER_ROW
_NBUF = 4


def _snap_body(bank_hbm, snap_hbm, buf, isem, osem):
    wid = lax.axis_index("s") * 2 + lax.axis_index("c")
    row0 = wid * _ROWS_PER_W

    def _in(j):
        row = row0 + j // _CHUNKS_PER_ROW
        col = (j % _CHUNKS_PER_ROW) * _CHUNK
        return pltpu.make_async_copy(
            bank_hbm.at[row, pl.ds(col, _CHUNK)], buf.at[j % _NBUF],
            isem.at[j % _NBUF])

    def _out(j):
        row = row0 + j // _CHUNKS_PER_ROW
        col = (j % _CHUNKS_PER_ROW) * _CHUNK
        return pltpu.make_async_copy(
            buf.at[j % _NBUF], snap_hbm.at[row, pl.ds(col, _CHUNK)],
            osem.at[j % _NBUF])

    for j in range(_NBUF):
        _in(j).start()
    for j in range(_TOTAL):
        _in(j).wait()
        _out(j).start()
        if j + _NBUF < _TOTAL:
            _out(j).wait()
            _in(j + _NBUF).start()
    for j in range(_TOTAL - _NBUF, _TOTAL):
        _out(j).wait()


@functools.partial(
    pl.kernel,
    out_type=jax.ShapeDtypeStruct((_DIM, _SIZE), jnp.float32),
    mesh=plsc.VectorSubcoreMesh(core_axis_name="c", subcore_axis_name="s"),
    scratch_types=[
        pltpu.VMEM((_NBUF, _CHUNK), jnp.float32),
        pltpu.SemaphoreType.DMA((_NBUF,)),
        pltpu.SemaphoreType.DMA((_NBUF,)),
    ],
)
def _snap_sc(bank_hbm, snap_hbm, buf, isem, osem):
    _snap_body(bank_hbm, snap_hbm, buf, isem, osem)


def _tc_body(out_in_ref, bank_ref, out_out_ref, new_ref):
    i = pl.program_id(0)
    new_ref[...] = bank_ref[...]

    @pl.when(i == 0)
    def _():
        out_out_ref[...] = out_in_ref[...]
        new_ref[:, :_BATCH] = jnp.transpose(out_in_ref[...])


def kernel(output, bank):
    snap = _snap_sc(bank)
    out, new = pl.pallas_call(
        _tc_body,
        grid=(_GRID,),
        in_specs=[
            pl.BlockSpec((_BATCH, _DIM), lambda i: (0, 0)),
            pl.BlockSpec((_DIM, _BLK), lambda i: (0, i)),
        ],
        out_specs=[
            pl.BlockSpec((_BATCH, _DIM), lambda i: (0, 0)),
            pl.BlockSpec((_DIM, _BLK), lambda i: (0, i)),
        ],
        out_shape=(
            jax.ShapeDtypeStruct((_BATCH, _DIM), output.dtype),
            jax.ShapeDtypeStruct((_DIM, _SIZE), bank.dtype),
        ),
    )(output, bank)
    return (out, snap, new)


# speedup vs baseline: 65.8752x; 1.9816x over previous
import jax
import jax.numpy as jnp
from jax.experimental import pallas as pl

_SIZE = 65536
_DIM = 128
_BATCH = 1024
_BLK = 16384
_GRID = _SIZE // _BLK


def _body(out_in_ref, bank_ref, out_out_ref, snap_ref, new_ref):
    i = pl.program_id(0)
    b = bank_ref[...]
    snap_ref[...] = b
    new_ref[...] = b

    @pl.when(i == 0)
    def _():
        out_out_ref[...] = out_in_ref[...]
        new_ref[:, :_BATCH] = jnp.transpose(out_in_ref[...])


def kernel(output, bank):
    out_shapes = (
        jax.ShapeDtypeStruct((_BATCH, _DIM), output.dtype),
        jax.ShapeDtypeStruct((_DIM, _SIZE), bank.dtype),
        jax.ShapeDtypeStruct((_DIM, _SIZE), bank.dtype),
    )
    out, snap, new = pl.pallas_call(
        _body,
        grid=(_GRID,),
        in_specs=[
            pl.BlockSpec((_BATCH, _DIM), lambda i: (0, 0)),
            pl.BlockSpec((_DIM, _BLK), lambda i: (0, i)),
        ],
        out_specs=[
            pl.BlockSpec((_BATCH, _DIM), lambda i: (0, 0)),
            pl.BlockSpec((_DIM, _BLK), lambda i: (0, i)),
            pl.BlockSpec((_DIM, _BLK), lambda i: (0, i)),
        ],
        out_shape=out_shapes,
    )(output, bank)
    return (out, snap, new)
